# Initial kernel scaffold; baseline (speedup 1.0000x reference)
import jax
import jax.numpy as jnp
from jax.experimental import pallas as pl


def kernel(x, length, emb_weight, proj_W, proj_b):
    raise NotImplementedError("write your pallas kernel here")


# speedup vs baseline: 1.3667x; 1.3667x over previous
import functools

import jax
import jax.numpy as jnp
from jax import lax
from jax.experimental import pallas as pl
from jax.experimental.pallas import tpu as pltpu
from jax.experimental.pallas import tpu_sc as plsc

_NC = 2
_NS = 16
_NW = _NC * _NS


def _proj_body(a_ref, b_ref, o_ref):
    o_ref[...] = jnp.dot(a_ref[...], b_ref[...],
                         preferred_element_type=jnp.float32)


def _project_table(emb_weight, proj_Wt):
    V, D = emb_weight.shape
    H = proj_Wt.shape[1]
    BM = 1000
    assert V % BM == 0
    return pl.pallas_call(
        _proj_body,
        grid=(V // BM,),
        in_specs=[
            pl.BlockSpec((BM, D), lambda i: (i, 0)),
            pl.BlockSpec((D, H), lambda i: (0, 0)),
        ],
        out_specs=pl.BlockSpec((BM, H), lambda i: (i, 0)),
        out_shape=jax.ShapeDtypeStruct((V, H), jnp.float32),
    )(emb_weight, proj_Wt)


def _fin_body(s_ref, l_ref, b_ref, o_ref):
    inv = 1.0 / l_ref[...].astype(jnp.float32)
    o_ref[...] = s_ref[...] * inv + b_ref[...]


def _finalize(sums, length, proj_b):
    B, H = sums.shape
    return pl.pallas_call(
        _fin_body,
        in_specs=[
            pl.BlockSpec((B, H), lambda: (0, 0)),
            pl.BlockSpec((B, 1), lambda: (0, 0)),
            pl.BlockSpec((1, H), lambda: (0, 0)),
        ],
        out_specs=pl.BlockSpec((B, H), lambda: (0, 0)),
        out_shape=jax.ShapeDtypeStruct((B, H), jnp.float32),
    )(sums, length.reshape(B, 1), proj_b.reshape(1, H))


def _make_sc_pool(B, H, CL, NCH):
    RB = B // _NW
    HV = H // 16

    def _accum(buf):
        def jbody(j, carry):
            return tuple(carry[k] + buf[j, pl.ds(16 * k, 16)]
                         for k in range(HV))
        init = tuple(jnp.zeros((16,), jnp.float32) for _ in range(HV))
        return lax.fori_loop(0, CL, jbody, init, unroll=4)

    @functools.partial(
        pl.kernel,
        mesh=plsc.VectorSubcoreMesh(core_axis_name="c", subcore_axis_name="s"),
        out_type=jax.ShapeDtypeStruct((B, H), jnp.float32),
        scratch_types=[
            pltpu.VMEM((RB, NCH, CL), jnp.int32),
            pltpu.VMEM((CL, H), jnp.float32),
            pltpu.VMEM((CL, H), jnp.float32),
            pltpu.VMEM((RB, H), jnp.float32),
            pltpu.SemaphoreType.DMA,
            pltpu.SemaphoreType.DMA,
        ],
    )
    def sc_pool(xp_hbm, table_hbm, sums_hbm, idx_v, buf0, buf1, out_v,
                sem0, sem1):
        wid = lax.axis_index("s") * _NC + lax.axis_index("c")
        base = wid * RB
        pltpu.sync_copy(xp_hbm.at[pl.ds(base, RB)], idx_v)
        pltpu.make_async_copy(table_hbm.at[idx_v.at[0, 0]], buf0, sem0).start()

        def row_body(b, carry):
            pltpu.make_async_copy(
                table_hbm.at[idx_v.at[b, 1]], buf1, sem1).start()
            pltpu.make_async_copy(
                table_hbm.at[idx_v.at[b, 0]], buf0, sem0).wait()
            acc0 = _accum(buf0)

            @pl.when(b + 1 < RB)
            def _():
                pltpu.make_async_copy(
                    table_hbm.at[idx_v.at[b + 1, 0]], buf0, sem0).start()

            pltpu.make_async_copy(
                table_hbm.at[idx_v.at[b, 1]], buf1, sem1).wait()
            acc1 = _accum(buf1)
            for k in range(HV):
                out_v[b, pl.ds(16 * k, 16)] = acc0[k] + acc1[k]
            return carry

        lax.fori_loop(0, RB, row_body, 0)
        pltpu.sync_copy(out_v, sums_hbm.at[pl.ds(base, RB)])

    return sc_pool


def kernel(x, length, emb_weight, proj_W, proj_b):
    B, L = x.shape
    H, D = proj_W.shape
    CL = 104
    NCH = 2
    pad = NCH * CL - L
    xi = x.astype(jnp.int32)
    xp = jnp.concatenate(
        [xi, jnp.full((B, pad), 1, jnp.int32)], axis=1).reshape(B, NCH, CL)

    emb_proj = _project_table(emb_weight, proj_W.T)
    sums = _make_sc_pool(B, H, CL, NCH)(xp, emb_proj)
    return _finalize(sums, length, proj_b)
